# baseline (speedup 1.0000x reference)
import functools

import jax
import jax.numpy as jnp
from jax import lax
from jax.experimental import pallas as pl
from jax.experimental.pallas import tpu as pltpu
from jax.experimental.pallas import tpu_sc as plsc

_FIELDS = 26
_FIELD_DIM = 40000
_BATCH = 16384
_NC, _NS, _L = 2, 16, 16
_NW = _NC * _NS
_ROWS_W = _BATCH // _NW
_LOOK_W = _ROWS_W * _FIELDS
_CHUNK = 128
_NDMA = _LOOK_W // _CHUNK


def _body(x_hbm, table_hbm, bias_hbm, out_hbm, xv, idxv, vals, outv, biasv, sem):
    wid = lax.axis_index("s") * _NC + lax.axis_index("c")
    base = wid * _LOOK_W

    pltpu.sync_copy(x_hbm.at[pl.ds(base, _LOOK_W)], xv)
    pltpu.sync_copy(bias_hbm, biasv)

    iota = lax.iota(jnp.int32, _L)
    bvec = biasv[pl.ds(0, _L)]

    def idx_body(j, carry):
        for l in range(_CHUNK // _L):
            p0 = j * _CHUNK + l * _L
            f = lax.rem(p0 + iota, _FIELDS)
            idxv[j, pl.ds(l * _L, _L)] = xv[pl.ds(p0, _L)] + f * _FIELD_DIM
        return carry

    lax.fori_loop(0, _NDMA, idx_body, 0)

    def fire(j, carry):
        pltpu.async_copy(
            table_hbm.at[idxv.at[j]],
            vals.at[j],
            sem,
        ).wait()
        return carry

    lax.fori_loop(0, _NDMA, fire, 0)

    def red_body(c, carry):
        b0 = c * _L
        pos0 = (b0 + iota) * _FIELDS
        acc = bvec
        for f in range(_FIELDS):
            p = pos0 + f
            row = lax.shift_right_logical(p, 7)
            col = lax.bitwise_and(p, _CHUNK - 1)
            acc = acc + plsc.load_gather(vals, [row, col])
        outv[pl.ds(b0, _L)] = acc
        return carry

    lax.fori_loop(0, _ROWS_W // _L, red_body, 0)

    pltpu.sync_copy(outv, out_hbm.at[pl.ds(wid * _ROWS_W, _ROWS_W)])


_fl_kernel = functools.partial(
    pl.kernel,
    out_type=jax.ShapeDtypeStruct((_BATCH,), jnp.float32),
    mesh=plsc.VectorSubcoreMesh(
        core_axis_name="c", subcore_axis_name="s",
        num_cores=_NC, num_subcores=_NS,
    ),
    scratch_types=[
        pltpu.VMEM((_LOOK_W,), jnp.int32),
        pltpu.VMEM((_NDMA, _CHUNK), jnp.int32),
        pltpu.VMEM((_NDMA, _CHUNK), jnp.float32),
        pltpu.VMEM((_ROWS_W,), jnp.float32),
        pltpu.VMEM((_L,), jnp.float32),
        pltpu.SemaphoreType.DMA,
    ],
    compiler_params=pltpu.CompilerParams(needs_layout_passes=False),
)(_body)


def kernel(x, table, bias):
    bias16 = jnp.broadcast_to(bias.reshape(()), (_L,)).astype(jnp.float32)
    out = _fl_kernel(x.reshape(-1), table.reshape(-1), bias16)
    return out.reshape(_BATCH, 1)


# speedup vs baseline: 1.6419x; 1.6419x over previous
import functools

import jax
import jax.numpy as jnp
from jax import lax
from jax.experimental import pallas as pl
from jax.experimental.pallas import tpu as pltpu
from jax.experimental.pallas import tpu_sc as plsc

_FIELDS = 26
_FIELD_DIM = 40000
_BATCH = 16384
_NC, _NS, _L = 2, 16, 16
_NW = _NC * _NS
_ROWS_W = _BATCH // _NW
_LOOK_W = _ROWS_W * _FIELDS
_CHUNK = 128
_NDMA = _LOOK_W // _CHUNK


def _body(x_hbm, table_hbm, bias_hbm, out_hbm, xv, idxv, vals, outv, biasv, sem):
    wid = lax.axis_index("s") * _NC + lax.axis_index("c")
    base = wid * _LOOK_W

    pltpu.sync_copy(x_hbm.at[pl.ds(base, _LOOK_W)], xv)
    pltpu.sync_copy(bias_hbm, biasv)

    iota = lax.iota(jnp.int32, _L)
    bvec = biasv[pl.ds(0, _L)]

    def idx_body(j, carry):
        for l in range(_CHUNK // _L):
            p0 = j * _CHUNK + l * _L
            f = lax.rem(p0 + iota, _FIELDS)
            idxv[j, pl.ds(l * _L, _L)] = xv[pl.ds(p0, _L)] + f * _FIELD_DIM
        return carry

    lax.fori_loop(0, _NDMA, idx_body, 0)

    def fire(j, carry):
        pltpu.async_copy(table_hbm.at[idxv.at[j]], vals.at[j], sem)
        return carry

    lax.fori_loop(0, _NDMA, fire, 0)

    def drain(j, carry):
        pltpu.make_async_copy(table_hbm.at[idxv.at[j]], vals.at[j], sem).wait()
        return carry

    lax.fori_loop(0, _NDMA, drain, 0)

    def red_body(c, carry):
        b0 = c * _L
        pos0 = (b0 + iota) * _FIELDS
        acc = bvec
        for f in range(_FIELDS):
            p = pos0 + f
            row = lax.shift_right_logical(p, 7)
            col = lax.bitwise_and(p, _CHUNK - 1)
            acc = acc + plsc.load_gather(vals, [row, col])
        outv[pl.ds(b0, _L)] = acc
        return carry

    lax.fori_loop(0, _ROWS_W // _L, red_body, 0)

    pltpu.sync_copy(outv, out_hbm.at[pl.ds(wid * _ROWS_W, _ROWS_W)])


_fl_kernel = functools.partial(
    pl.kernel,
    out_type=jax.ShapeDtypeStruct((_BATCH,), jnp.float32),
    mesh=plsc.VectorSubcoreMesh(
        core_axis_name="c", subcore_axis_name="s",
        num_cores=_NC, num_subcores=_NS,
    ),
    scratch_types=[
        pltpu.VMEM((_LOOK_W,), jnp.int32),
        pltpu.VMEM((_NDMA, _CHUNK), jnp.int32),
        pltpu.VMEM((_NDMA, _CHUNK), jnp.float32),
        pltpu.VMEM((_ROWS_W,), jnp.float32),
        pltpu.VMEM((_L,), jnp.float32),
        pltpu.SemaphoreType.DMA,
    ],
    compiler_params=pltpu.CompilerParams(needs_layout_passes=False),
)(_body)


def kernel(x, table, bias):
    bias16 = jnp.broadcast_to(bias.reshape(()), (_L,)).astype(jnp.float32)
    out = _fl_kernel(x.reshape(-1), table.reshape(-1), bias16)
    return out.reshape(_BATCH, 1)
